# Initial kernel scaffold; baseline (speedup 1.0000x reference)
import jax
import jax.numpy as jnp
from jax.experimental import pallas as pl


def kernel(x, W1, b1, W2, b2, Wfc2, bfc2):
    raise NotImplementedError("write your pallas kernel here")


# speedup vs baseline: 17.3006x; 17.3006x over previous
import numpy as np

import jax
import jax.numpy as jnp
from jax.experimental import pallas as pl
from jax.experimental.pallas import tpu as pltpu

_K = 3


def _shift_mats(w):
    s = np.zeros((_K, w, w), np.float32)
    b = np.arange(w)
    for j in range(_K):
        s[j, (b + j - 1) % w, b] = 1.0
    return s


def _col_pool_select(c, w):
    m = np.zeros((c * w, c * (w // 2)), np.float32)
    q = np.arange(w // 2)
    for co in range(c):
        m[co * w + 2 * q, co * (w // 2) + q] = 1.0
    return m


def _fwd_kernel(Nb, H, W, cin, c1, c2, ncls,
                x_ref, a1_ref, a2_ref, s1_ref, s2_ref, wfc_ref,
                b1_ref, b2_ref, bfc_ref, out_ref):
    f32 = jnp.float32
    h2, w2 = H // 2, W // 2
    h3, w3 = h2 // 2, w2 // 2
    nc1, nc2 = c1 * W, c2 * w2

    xb = x_ref[...].reshape(H * Nb, cin * W)
    z1 = jnp.zeros((Nb, cin * W), f32)
    xf = jnp.concatenate([z1, xb, z1], axis=0)
    y = (jnp.dot(xf[0:H * Nb], a1_ref[0], preferred_element_type=f32)
         + jnp.dot(xf[Nb:(H + 1) * Nb], a1_ref[1], preferred_element_type=f32)
         + jnp.dot(xf[2 * Nb:(H + 2) * Nb], a1_ref[2], preferred_element_type=f32)
         + b1_ref[...])

    y3 = y.reshape(h2, 2, Nb, nc1)
    yr = jnp.maximum(y3[:, 0], y3[:, 1])
    yc = jnp.maximum(yr, pltpu.roll(yr, nc1 - 1, axis=2))
    p1 = jnp.maximum(
        jnp.dot(yc.reshape(h2 * Nb, nc1), s1_ref[...],
                preferred_element_type=f32), 0.0)

    z2 = jnp.zeros((Nb, c1 * w2), f32)
    pf = jnp.concatenate([z2, p1, z2], axis=0)
    y2 = (jnp.dot(pf[0:h2 * Nb], a2_ref[0], preferred_element_type=f32)
          + jnp.dot(pf[Nb:(h2 + 1) * Nb], a2_ref[1], preferred_element_type=f32)
          + jnp.dot(pf[2 * Nb:(h2 + 2) * Nb], a2_ref[2], preferred_element_type=f32)
          + b2_ref[...])

    y23 = y2.reshape(h3, 2, Nb, nc2)
    y2r = jnp.maximum(y23[:, 0], y23[:, 1])
    y2c = jnp.maximum(y2r, pltpu.roll(y2r, nc2 - 1, axis=2))
    p2 = jnp.maximum(
        jnp.dot(y2c.reshape(h3 * Nb, nc2), s2_ref[...],
                preferred_element_type=f32), 0.0)

    p23 = p2.reshape(h3, Nb, c2 * w3)
    acc = jnp.dot(p23[0], wfc_ref[0], preferred_element_type=f32)
    for r in range(1, h3):
        acc = acc + jnp.dot(p23[r], wfc_ref[r], preferred_element_type=f32)
    out_ref[...] = acc + bfc_ref[...]


def kernel(x, W1, b1, W2, b2, Wfc2, bfc2):
    import functools
    f32 = jnp.float32
    B, cin, H, W = x.shape
    c1 = W1.shape[0]
    c2 = W2.shape[0]
    ncls = Wfc2.shape[0]
    h2, w2 = H // 2, W // 2
    h3, w3 = h2 // 2, w2 // 2

    Nb = 32
    while B % Nb:
        Nb //= 2
    G = B // Nb

    xt = x.astype(f32).transpose(2, 0, 1, 3).reshape(H, B, cin * W)

    sm1 = _shift_mats(W)
    sm2 = _shift_mats(w2)
    A1 = jnp.einsum('ocdj,jab->dcaob', W1.astype(f32),
                    jnp.asarray(sm1)).reshape(_K, cin * W, c1 * W)
    A2 = jnp.einsum('ocdj,jab->dcaob', W2.astype(f32),
                    jnp.asarray(sm2)).reshape(_K, c1 * w2, c2 * w2)

    S1 = jnp.asarray(_col_pool_select(c1, W))
    S2 = jnp.asarray(_col_pool_select(c2, w2))
    Wfc3 = (Wfc2.astype(f32).reshape(ncls, c2, h3, w3)
            .transpose(2, 1, 3, 0).reshape(h3, c2 * w3, ncls))
    b1r = jnp.repeat(b1.astype(f32), W).reshape(1, c1 * W)
    b2r = jnp.repeat(b2.astype(f32), w2).reshape(1, c2 * w2)
    bfc = bfc2.astype(f32).reshape(1, ncls)

    kfn = functools.partial(_fwd_kernel, Nb, H, W, cin, c1, c2, ncls)

    def const_spec(a):
        nd = a.ndim
        return pl.BlockSpec(a.shape, lambda g, _n=nd: (0,) * _n)

    out = pl.pallas_call(
        kfn,
        out_shape=jax.ShapeDtypeStruct((B, ncls), f32),
        grid=(G,),
        in_specs=[
            pl.BlockSpec((H, Nb, cin * W), lambda g: (0, g, 0)),
            const_spec(A1), const_spec(A2),
            const_spec(S1), const_spec(S2), const_spec(Wfc3),
            const_spec(b1r), const_spec(b2r), const_spec(bfc),
        ],
        out_specs=pl.BlockSpec((Nb, ncls), lambda g: (g, 0)),
        compiler_params=pltpu.CompilerParams(
            dimension_semantics=("parallel",)),
    )(xt, A1, A2, S1, S2, Wfc3, b1r, b2r, bfc)
    return out


# speedup vs baseline: 20.1712x; 1.1659x over previous
import numpy as np

import jax
import jax.numpy as jnp
from jax.experimental import pallas as pl
from jax.experimental.pallas import tpu as pltpu

_K = 3


def _shift_mats(w):
    s = np.zeros((_K, w, w), np.float32)
    b = np.arange(w)
    for j in range(_K):
        s[j, (b + j - 1) % w, b] = 1.0
    return s


def _col_pool_select(c, w):
    m = np.zeros((c * w, c * (w // 2)), np.float32)
    q = np.arange(w // 2)
    for co in range(c):
        m[co * w + 2 * q, co * (w // 2) + q] = 1.0
    return m


def _fwd_kernel(Nb, H, W, cin, c1, c2, ncls,
                x_ref, a1_ref, a2_ref, s1_ref, s2_ref, wfc_ref,
                b1_ref, b2_ref, bfc_ref, out_ref):
    f32 = jnp.float32
    bf16 = jnp.bfloat16
    h2, w2 = H // 2, W // 2
    h3, w3 = h2 // 2, w2 // 2
    nc1, nc2 = c1 * W, c2 * w2

    xb = x_ref[...].reshape(H * Nb, cin * W)
    z1 = jnp.zeros((Nb, cin * W), bf16)
    xf = jnp.concatenate([z1, xb, z1], axis=0)
    y = (jnp.dot(xf[0:H * Nb], a1_ref[0], preferred_element_type=f32)
         + jnp.dot(xf[Nb:(H + 1) * Nb], a1_ref[1], preferred_element_type=f32)
         + jnp.dot(xf[2 * Nb:(H + 2) * Nb], a1_ref[2], preferred_element_type=f32)
         + b1_ref[...]).astype(bf16)

    y3 = y.reshape(h2, 2, Nb, nc1)
    yr = jnp.maximum(y3[:, 0], y3[:, 1])
    yc = jnp.maximum(yr, pltpu.roll(yr, nc1 - 1, axis=2))
    p1 = jnp.maximum(
        jnp.dot(yc.reshape(h2 * Nb, nc1), s1_ref[...],
                preferred_element_type=f32), 0.0).astype(bf16)

    z2 = jnp.zeros((Nb, c1 * w2), bf16)
    pf = jnp.concatenate([z2, p1, z2], axis=0)
    y2 = (jnp.dot(pf[0:h2 * Nb], a2_ref[0], preferred_element_type=f32)
          + jnp.dot(pf[Nb:(h2 + 1) * Nb], a2_ref[1], preferred_element_type=f32)
          + jnp.dot(pf[2 * Nb:(h2 + 2) * Nb], a2_ref[2], preferred_element_type=f32)
          + b2_ref[...]).astype(bf16)

    y23 = y2.reshape(h3, 2, Nb, nc2)
    y2r = jnp.maximum(y23[:, 0], y23[:, 1])
    y2c = jnp.maximum(y2r, pltpu.roll(y2r, nc2 - 1, axis=2))
    p2 = jnp.maximum(
        jnp.dot(y2c.reshape(h3 * Nb, nc2), s2_ref[...],
                preferred_element_type=f32), 0.0).astype(bf16)

    p23 = p2.reshape(h3, Nb, c2 * w3)
    acc = jnp.dot(p23[0], wfc_ref[0], preferred_element_type=f32)
    for r in range(1, h3):
        acc = acc + jnp.dot(p23[r], wfc_ref[r], preferred_element_type=f32)
    out_ref[...] = acc + bfc_ref[...]


def kernel(x, W1, b1, W2, b2, Wfc2, bfc2):
    import functools
    f32 = jnp.float32
    bf16 = jnp.bfloat16
    B, cin, H, W = x.shape
    c1 = W1.shape[0]
    c2 = W2.shape[0]
    ncls = Wfc2.shape[0]
    h2, w2 = H // 2, W // 2
    h3, w3 = h2 // 2, w2 // 2

    Nb = 32
    while B % Nb:
        Nb //= 2
    G = B // Nb

    xt = x.astype(bf16).transpose(2, 0, 1, 3).reshape(H, B, cin * W)

    sm1 = _shift_mats(W)
    sm2 = _shift_mats(w2)
    A1 = jnp.einsum('ocdj,jab->dcaob', W1.astype(f32),
                    jnp.asarray(sm1)).reshape(_K, cin * W, c1 * W).astype(bf16)
    A2 = jnp.einsum('ocdj,jab->dcaob', W2.astype(f32),
                    jnp.asarray(sm2)).reshape(_K, c1 * w2, c2 * w2).astype(bf16)

    S1 = jnp.asarray(_col_pool_select(c1, W), bf16)
    S2 = jnp.asarray(_col_pool_select(c2, w2), bf16)
    Wfc3 = (Wfc2.astype(f32).reshape(ncls, c2, h3, w3)
            .transpose(2, 1, 3, 0).reshape(h3, c2 * w3, ncls).astype(bf16))
    b1r = jnp.repeat(b1.astype(f32), W).reshape(1, c1 * W)
    b2r = jnp.repeat(b2.astype(f32), w2).reshape(1, c2 * w2)
    bfc = bfc2.astype(f32).reshape(1, ncls)

    kfn = functools.partial(_fwd_kernel, Nb, H, W, cin, c1, c2, ncls)

    def const_spec(a):
        nd = a.ndim
        return pl.BlockSpec(a.shape, lambda g, _n=nd: (0,) * _n)

    out = pl.pallas_call(
        kfn,
        out_shape=jax.ShapeDtypeStruct((B, ncls), f32),
        grid=(G,),
        in_specs=[
            pl.BlockSpec((H, Nb, cin * W), lambda g: (0, g, 0)),
            const_spec(A1), const_spec(A2),
            const_spec(S1), const_spec(S2), const_spec(Wfc3),
            const_spec(b1r), const_spec(b2r), const_spec(bfc),
        ],
        out_specs=pl.BlockSpec((Nb, ncls), lambda g: (g, 0)),
        compiler_params=pltpu.CompilerParams(
            dimension_semantics=("parallel",)),
    )(xt, A1, A2, S1, S2, Wfc3, b1r, b2r, bfc)
    return out


# speedup vs baseline: 24.5045x; 1.2148x over previous
import numpy as np

import jax
import jax.numpy as jnp
from jax.experimental import pallas as pl
from jax.experimental.pallas import tpu as pltpu

_K = 3


def _shift_mats(w):
    s = np.zeros((_K, w, w), np.float32)
    b = np.arange(w)
    for j in range(_K):
        s[j, (b + j - 1) % w, b] = 1.0
    return s


def _col_pool_select(c, w):
    m = np.zeros((c * w, c * (w // 2)), np.float32)
    q = np.arange(w // 2)
    for co in range(c):
        m[co * w + 2 * q, co * (w // 2) + q] = 1.0
    return m


def _fwd_kernel(Nb, H, W, cin, c1, c2, ncls,
                x_ref, a1_ref, a2_ref, s1_ref, s2_ref, wfc_ref,
                b1_ref, b2_ref, bfc_ref, out_ref):
    f32 = jnp.float32
    bf16 = jnp.bfloat16
    h2, w2 = H // 2, W // 2
    h3, w3 = h2 // 2, w2 // 2
    nc1, nc2 = c1 * W, c2 * w2

    kp = 128 * ((cin * W + 127) // 128)
    xb = x_ref[...].reshape(H * Nb, cin * W)
    z1 = jnp.zeros((Nb, cin * W), bf16)
    xf = jnp.concatenate([z1, xb, z1], axis=0)
    xc = jnp.concatenate(
        [jnp.pad(xf[d * Nb:(H + d) * Nb], ((0, 0), (0, kp - cin * W)))
         for d in range(_K)], axis=1)
    y = (jnp.dot(xc, a1_ref[...], preferred_element_type=f32)
         + b1_ref[...]).astype(bf16)

    y3 = y.reshape(h2, 2, Nb, nc1)
    yr = jnp.maximum(y3[:, 0], y3[:, 1])
    yc = jnp.maximum(yr, pltpu.roll(yr, nc1 - 1, axis=2))
    p1 = jnp.maximum(
        jnp.dot(yc.reshape(h2 * Nb, nc1), s1_ref[...],
                preferred_element_type=f32), 0.0).astype(bf16)

    z2 = jnp.zeros((Nb, c1 * w2), bf16)
    pf = jnp.concatenate([z2, p1, z2], axis=0)
    pc = jnp.concatenate(
        [pf[d * Nb:(h2 + d) * Nb] for d in range(_K)], axis=1)
    y2 = (jnp.dot(pc, a2_ref[...], preferred_element_type=f32)
          + b2_ref[...]).astype(bf16)

    y23 = y2.reshape(h3, 2, Nb, nc2)
    y2r = jnp.maximum(y23[:, 0], y23[:, 1])
    y2c = jnp.maximum(y2r, pltpu.roll(y2r, nc2 - 1, axis=2))
    p2 = jnp.maximum(
        jnp.dot(y2c.reshape(h3 * Nb, nc2), s2_ref[...],
                preferred_element_type=f32), 0.0).astype(bf16)

    p23 = p2.reshape(h3, Nb, c2 * w3)
    acc = jnp.dot(p23[0], wfc_ref[0], preferred_element_type=f32)
    for r in range(1, h3):
        acc = acc + jnp.dot(p23[r], wfc_ref[r], preferred_element_type=f32)
    out_ref[...] = acc + bfc_ref[...]


def kernel(x, W1, b1, W2, b2, Wfc2, bfc2):
    import functools
    f32 = jnp.float32
    bf16 = jnp.bfloat16
    B, cin, H, W = x.shape
    c1 = W1.shape[0]
    c2 = W2.shape[0]
    ncls = Wfc2.shape[0]
    h2, w2 = H // 2, W // 2
    h3, w3 = h2 // 2, w2 // 2

    Nb = 64
    while B % Nb:
        Nb //= 2
    G = B // Nb

    xt = x.astype(bf16).transpose(2, 0, 1, 3).reshape(H, B, cin * W)

    sm1 = _shift_mats(W)
    sm2 = _shift_mats(w2)
    kp = 128 * ((cin * W + 127) // 128)
    A1 = jnp.einsum('ocdj,jab->dcaob', W1.astype(f32),
                    jnp.asarray(sm1)).reshape(_K, cin * W, c1 * W)
    A1 = jnp.pad(A1, ((0, 0), (0, kp - cin * W), (0, 0))
                 ).reshape(_K * kp, c1 * W).astype(bf16)
    A2 = jnp.einsum('ocdj,jab->dcaob', W2.astype(f32),
                    jnp.asarray(sm2)).reshape(_K * c1 * w2, c2 * w2).astype(bf16)

    S1 = jnp.asarray(_col_pool_select(c1, W), bf16)
    S2 = jnp.asarray(_col_pool_select(c2, w2), bf16)
    Wfc3 = (Wfc2.astype(f32).reshape(ncls, c2, h3, w3)
            .transpose(2, 1, 3, 0).reshape(h3, c2 * w3, ncls).astype(bf16))
    b1r = jnp.repeat(b1.astype(f32), W).reshape(1, c1 * W)
    b2r = jnp.repeat(b2.astype(f32), w2).reshape(1, c2 * w2)
    bfc = bfc2.astype(f32).reshape(1, ncls)

    kfn = functools.partial(_fwd_kernel, Nb, H, W, cin, c1, c2, ncls)

    def const_spec(a):
        nd = a.ndim
        return pl.BlockSpec(a.shape, lambda g, _n=nd: (0,) * _n)

    out = pl.pallas_call(
        kfn,
        out_shape=jax.ShapeDtypeStruct((B, ncls), f32),
        grid=(G,),
        in_specs=[
            pl.BlockSpec((H, Nb, cin * W), lambda g: (0, g, 0)),
            const_spec(A1), const_spec(A2),
            const_spec(S1), const_spec(S2), const_spec(Wfc3),
            const_spec(b1r), const_spec(b2r), const_spec(bfc),
        ],
        out_specs=pl.BlockSpec((Nb, ncls), lambda g: (g, 0)),
        compiler_params=pltpu.CompilerParams(
            dimension_semantics=("parallel",)),
    )(xt, A1, A2, S1, S2, Wfc3, b1r, b2r, bfc)
    return out
